# baseline (speedup 1.0000x reference)
import functools

import jax
import jax.numpy as jnp
from jax import lax
from jax.experimental import pallas as pl
from jax.experimental.pallas import tpu as pltpu
from jax.experimental.pallas import tpu_sc as plsc

B = 16384
EMBED = 16
NC, NS = 2, 16
NW = NC * NS
CHUNK = 128
UC = B // (NW * CHUNK)
IC = 2 * B // (NW * CHUNK)
RB = 2048
NBLK = B // RB


def _sc_gather(user_table, item_table, uidx, iidx):
    mesh = plsc.VectorSubcoreMesh(core_axis_name="c", subcore_axis_name="s")

    @functools.partial(
        pl.kernel,
        out_type=(
            jax.ShapeDtypeStruct((NW * UC, CHUNK, EMBED), jnp.float32),
            jax.ShapeDtypeStruct((NW * IC, CHUNK, EMBED), jnp.float32),
        ),
        mesh=mesh,
        compiler_params=pltpu.CompilerParams(use_tc_tiling_on_sc=False),
        scratch_types=[
            pltpu.VMEM((UC, CHUNK), jnp.int32),
            pltpu.VMEM((IC, CHUNK), jnp.int32),
            pltpu.VMEM((UC, CHUNK, EMBED), jnp.float32),
            pltpu.VMEM((IC, CHUNK, EMBED), jnp.float32),
            pltpu.SemaphoreType.DMA,
        ],
    )
    def gather(ut_hbm, it_hbm, uidx_hbm, iidx_hbm, uout, iout,
               uidx_v, iidx_v, urows_v, irows_v, sem):
        wid = lax.axis_index("s") * NC + lax.axis_index("c")
        pltpu.sync_copy(uidx_hbm.at[pl.ds(wid * UC, UC)], uidx_v)
        pltpu.sync_copy(iidx_hbm.at[pl.ds(wid * IC, IC)], iidx_v)
        copies = []
        for j in range(UC):
            copies.append(pltpu.async_copy(ut_hbm.at[uidx_v.at[j]], urows_v.at[j], sem))
        for j in range(IC):
            copies.append(pltpu.async_copy(it_hbm.at[iidx_v.at[j]], irows_v.at[j], sem))
        for c in copies:
            c.wait()
        pltpu.sync_copy(urows_v, uout.at[pl.ds(wid * UC, UC)])
        pltpu.sync_copy(irows_v, iout.at[pl.ds(wid * IC, IC)])

    return gather(user_table, item_table, uidx, iidx)


def _mlp_body(ue_ref, pe_ref, ne_ref,
              uw0, ub0, uw1, ub1, uw2, ub2, uw3, ub3,
              iw0, ib0, iw1, ib1, iw2, ib2, iw3, ib3,
              dw, db, out_ref):
    u = ue_ref[...]
    for W, b in ((uw0, ub0), (uw1, ub1), (uw2, ub2), (uw3, ub3)):
        u = jnp.maximum(jnp.dot(u, W[...], preferred_element_type=jnp.float32) + b[...], 0.0)
    p = pe_ref[...]
    n = ne_ref[...]
    for W, b in ((iw0, ib0), (iw1, ib1), (iw2, ib2), (iw3, ib3)):
        Wv, bv = W[...], b[...]
        p = jnp.maximum(jnp.dot(p, Wv, preferred_element_type=jnp.float32) + bv, 0.0)
        n = jnp.maximum(jnp.dot(n, Wv, preferred_element_type=jnp.float32) + bv, 0.0)
    w = dw[...]
    bias = db[...]
    pv = jax.nn.sigmoid(u * p)
    nv = jax.nn.sigmoid(u * n)
    pos_l = jnp.sum(pv * w, axis=1, keepdims=True) + bias
    neg_l = jnp.sum(nv * w, axis=1, keepdims=True) + bias
    out_ref[...] = jnp.concatenate([pos_l, neg_l], axis=1)


def _tc_mlp(ue, irows, weights):
    def wspec(w):
        if w.ndim == 2:
            return pl.BlockSpec(w.shape, lambda i: (0, 0))
        raise ValueError(w.shape)

    in_specs = [
        pl.BlockSpec((RB, EMBED), lambda i: (i, 0)),
        pl.BlockSpec((RB, EMBED), lambda i: (i, 0)),
        pl.BlockSpec((RB, EMBED), lambda i: (i + NBLK, 0)),
    ] + [wspec(w) for w in weights]

    return pl.pallas_call(
        _mlp_body,
        grid=(NBLK,),
        in_specs=in_specs,
        out_specs=pl.BlockSpec((RB, 2), lambda i: (i, 0)),
        out_shape=jax.ShapeDtypeStruct((B, 2), jnp.float32),
    )(ue, irows, irows, *weights)


def kernel(user, pos, neg, user_table, item_table,
           uW0, ub0, uW1, ub1, uW2, ub2, uW3, ub3,
           iW0, ib0, iW1, ib1, iW2, ib2, iW3, ib3,
           dW, db):
    uidx = user.reshape(B // CHUNK, CHUNK).astype(jnp.int32)
    iidx = jnp.concatenate([pos.reshape(-1), neg.reshape(-1)]).reshape(
        2 * B // CHUNK, CHUNK).astype(jnp.int32)

    urows, irows = _sc_gather(user_table, item_table, uidx, iidx)
    ue = urows.reshape(B, EMBED)
    irows = irows.reshape(2 * B, EMBED)

    weights = (
        uW0, ub0.reshape(1, -1), uW1, ub1.reshape(1, -1),
        uW2, ub2.reshape(1, -1), uW3, ub3.reshape(1, -1),
        iW0, ib0.reshape(1, -1), iW1, ib1.reshape(1, -1),
        iW2, ib2.reshape(1, -1), iW3, ib3.reshape(1, -1),
        dW.reshape(1, -1), db.reshape(1, 1),
    )
    return _tc_mlp(ue, irows, weights)
